# baseline (speedup 1.0000x reference)
import functools

import jax
import jax.numpy as jnp
from jax import lax
from jax.experimental import pallas as pl
from jax.experimental.pallas import tpu as pltpu
from jax.experimental.pallas import tpu_sc as plsc

N = 10000
E = 160000
D = 256
HD = 128
NSTEPS = 3
NETYPES = 3
NC = 2
NS = 16
EPT = E // NS
EPTP = 10240
CHUNK = 320
NCHUNK = EPTP // CHUNK
NPAD = 10240
RPT = NPAD // NS
BPW = NPAD // (NC * NS)
NB = 10
BLK = N // NB

@functools.cache
def _mesh():
    return plsc.VectorSubcoreMesh(
        core_axis_name="c", subcore_axis_name="s",
        num_cores=NC, num_subcores=NS)


def _emb_body(tok_hbm, table_hbm, out_hbm, tok_v, rows_v, sem):
    c = lax.axis_index("c")
    s = lax.axis_index("s")
    base = (s * NC + c) * BPW
    pltpu.sync_copy(tok_hbm.at[pl.ds(base, BPW)], tok_v)
    pltpu.async_copy(table_hbm.at[tok_v], rows_v, sem).wait()
    pltpu.sync_copy(rows_v, out_hbm.at[pl.ds(base, BPW)])


@functools.cache
def _emb_gather_kernel():
    return pl.kernel(
        _emb_body,
        out_type=jax.ShapeDtypeStruct((NPAD, D), jnp.float32),
        mesh=_mesh(),
        scratch_types=[
            pltpu.VMEM((BPW,), jnp.int32),
            pltpu.VMEM((BPW, D), jnp.float32),
            pltpu.SemaphoreType.DMA,
        ],
    )


def _edge_body(t_hbm, gidx_hbm, dst_hbm, out_hbm,
               gbuf, dbuf, rows, a_sh, sem):
    c = lax.axis_index("c")
    s = lax.axis_index("s")
    w = c * NS + s

    def zrow(i, carry):
        for j in range(HD // 16):
            rows[i, pl.ds(j * 16, 16)] = jnp.zeros((16,), jnp.float32)
        return carry

    lax.fori_loop(0, 128, zrow, 0)
    for k in range(RPT // 128):
        pltpu.sync_copy(rows.at[pl.ds(0, 128)],
                        a_sh.at[pl.ds(s * RPT + k * 128, 128)])
    plsc.subcore_barrier()

    def chunk(j, carry):
        pltpu.sync_copy(gidx_hbm.at[pl.ds(w * EPTP + j * CHUNK, CHUNK)], gbuf)
        pltpu.sync_copy(dst_hbm.at[pl.ds(s * EPTP + j * CHUNK, CHUNK)], dbuf)
        pltpu.async_copy(t_hbm.at[gbuf], rows, sem).wait()
        pltpu.sync_copy(rows, a_sh.at[dbuf], add=True)
        return carry

    lax.fori_loop(0, NCHUNK, chunk, 0)

    plsc.subcore_barrier()
    pltpu.sync_copy(a_sh.at[pl.ds(s * RPT, RPT)],
                    out_hbm.at[c, pl.ds(s * RPT, RPT)])


@functools.cache
def _edge_msg_kernel():
    return pl.kernel(
        _edge_body,
        out_type=jax.ShapeDtypeStruct((NC, NPAD, HD), jnp.float32),
        mesh=_mesh(),
        scratch_types=[
            pltpu.VMEM((CHUNK,), jnp.int32),
            pltpu.VMEM((CHUNK,), jnp.int32),
            pltpu.VMEM((CHUNK, HD), jnp.float32),
            pltpu.VMEM_SHARED((NPAD, HD), jnp.float32),
            pltpu.SemaphoreType.DMA,
        ],
    )


def _etype_mm_body(h_ref, wet_ref, out_ref):
    out_ref[0, 0] = jnp.dot(h_ref[...], wet_ref[0],
                            preferred_element_type=jnp.float32)


def _etype_mm(h, wet):
    return pl.pallas_call(
        _etype_mm_body,
        grid=(NB, NC, NETYPES),
        in_specs=[
            pl.BlockSpec((BLK, D), lambda nb, c, e: (nb, 0)),
            pl.BlockSpec((1, D, HD), lambda nb, c, e: (e, 0, c)),
        ],
        out_specs=pl.BlockSpec((1, 1, BLK, HD), lambda nb, c, e: (c, e, nb, 0)),
        out_shape=jax.ShapeDtypeStruct((NC, NETYPES, N, HD), jnp.float32),
    )(h, wet)


def _gru_body(a_ref, h_ref, wih_ref, whh_ref, bih_ref, bhh_ref, out_ref):
    h = h_ref[...]
    gi = (jnp.dot(a_ref[0], wih_ref[0], preferred_element_type=jnp.float32)
          + jnp.dot(a_ref[1], wih_ref[1], preferred_element_type=jnp.float32)
          + bih_ref[0][None, :])
    gh = (jnp.dot(h, whh_ref[...], preferred_element_type=jnp.float32)
          + bhh_ref[0][None, :])
    i_r, i_z, i_n = gi[:, :D], gi[:, D:2 * D], gi[:, 2 * D:]
    h_r, h_z, h_n = gh[:, :D], gh[:, D:2 * D], gh[:, 2 * D:]
    r = 1.0 / (1.0 + jnp.exp(-(i_r + h_r)))
    z = 1.0 / (1.0 + jnp.exp(-(i_z + h_z)))
    n = jnp.tanh(i_n + r * h_n)
    out_ref[...] = (1.0 - z) * n + z * h


def _gru(a, h, wih2, whh, bih, bhh):
    return pl.pallas_call(
        _gru_body,
        grid=(NB,),
        in_specs=[
            pl.BlockSpec((NC, BLK, HD), lambda nb: (0, nb, 0)),
            pl.BlockSpec((BLK, D), lambda nb: (nb, 0)),
            pl.BlockSpec((NC, HD, 3 * D), lambda nb: (0, 0, 0)),
            pl.BlockSpec((D, 3 * D), lambda nb: (0, 0)),
            pl.BlockSpec((1, 3 * D), lambda nb: (0, 0)),
            pl.BlockSpec((1, 3 * D), lambda nb: (0, 0)),
        ],
        out_specs=pl.BlockSpec((BLK, D), lambda nb: (nb, 0)),
        out_shape=jax.ShapeDtypeStruct((N, D), jnp.float32),
    )(a, h, wih2, whh, bih, bhh)


def _bn_stats_body(h_ref, e_ref, out_ref):
    @pl.when(pl.program_id(0) == 0)
    def _init():
        out_ref[...] = jnp.zeros_like(out_ref)

    hb = h_ref[...]
    eb = e_ref[...]
    out_ref[0, :D] += jnp.sum(hb, axis=0)
    out_ref[0, D:] += jnp.sum(eb, axis=0)
    out_ref[1, :D] += jnp.sum(hb * hb, axis=0)
    out_ref[1, D:] += jnp.sum(eb * eb, axis=0)


def _bn_stats(h, emb):
    return pl.pallas_call(
        _bn_stats_body,
        grid=(NB,),
        in_specs=[
            pl.BlockSpec((BLK, D), lambda nb: (nb, 0)),
            pl.BlockSpec((BLK, D), lambda nb: (nb, 0)),
        ],
        out_specs=pl.BlockSpec((2, 2 * D), lambda nb: (0, 0)),
        out_shape=jax.ShapeDtypeStruct((2, 2 * D), jnp.float32),
    )(h, emb)


def _bn_mean_body(h_ref, e_ref, mv_ref, gb_ref, out_ref):
    @pl.when(pl.program_id(0) == 0)
    def _init():
        out_ref[...] = jnp.zeros_like(out_ref)

    inv = lax.rsqrt(mv_ref[1] + 1e-5)
    mean = mv_ref[0]
    xh = (h_ref[...] - mean[:D][None, :]) * inv[:D][None, :]
    xe = (e_ref[...] - mean[D:][None, :]) * inv[D:][None, :]
    out_ref[0, :D] += jnp.sum(xh, axis=0)
    out_ref[0, D:] += jnp.sum(xe, axis=0)

    @pl.when(pl.program_id(0) == NB - 1)
    def _fin():
        out_ref[0, :] = (out_ref[0, :] * gb_ref[0] * (1.0 / N)) + gb_ref[1]


def _bn_mean(h, emb, mv, gb):
    return pl.pallas_call(
        _bn_mean_body,
        grid=(NB,),
        in_specs=[
            pl.BlockSpec((BLK, D), lambda nb: (nb, 0)),
            pl.BlockSpec((BLK, D), lambda nb: (nb, 0)),
            pl.BlockSpec((2, 2 * D), lambda nb: (0, 0)),
            pl.BlockSpec((2, 2 * D), lambda nb: (0, 0)),
        ],
        out_specs=pl.BlockSpec((1, 2 * D), lambda nb: (0, 0)),
        out_shape=jax.ShapeDtypeStruct((1, 2 * D), jnp.float32),
    )(h, emb, mv, gb)


def _head_body(f1_ref, f2_ref, fw_ref, fb_ref, out_ref):
    d = f1_ref[0, :] - f2_ref[0, :]
    euc = d * d
    logits = jnp.sum(euc[None, :] * fw_ref[...], axis=1) + fb_ref[0, :]
    act = jnp.where(logits >= 0, logits, 0.01 * logits)
    m = jnp.max(act)
    e = jnp.exp(act - m)
    out_ref[0, :] = e / jnp.sum(e)


def _head(f1, f2, fW, fb):
    return pl.pallas_call(
        _head_body,
        out_shape=jax.ShapeDtypeStruct((1, fW.shape[0]), jnp.float32),
    )(f1, f2, fW, fb.reshape(1, -1))


def _prep(tokens, edge_index, edge_type, We, Wih, Whh, bih, bhh):
    tok_pad = jnp.concatenate(
        [tokens.astype(jnp.int32), jnp.zeros((NPAD - N,), jnp.int32)])
    src = edge_index[0].astype(jnp.int32)
    pad = ((0, 0), (0, EPTP - EPT))
    dst = jnp.pad(edge_index[1].astype(jnp.int32).reshape(NS, EPT), pad,
                  constant_values=N).reshape(-1)
    gidx = jnp.pad((edge_type.astype(jnp.int32) * N + src).reshape(NS, EPT),
                   pad).reshape(-1)
    gidx2 = jnp.concatenate([gidx, gidx + NETYPES * N])
    wet = We.transpose(0, 2, 1)
    wih2 = Wih.T.reshape(NC, HD, 3 * D)
    whh = Whh.T
    return tok_pad, gidx2, dst, wet, wih2, whh, bih.reshape(1, -1), bhh.reshape(1, -1)


def _ggnn_feats(tokens, edge_index, edge_type, emb_table, We, Wih, Whh,
                bih, bhh, gamma, beta):
    tok_pad, gidx2, dst, wet, wih2, whh, bih2, bhh2 = _prep(
        tokens, edge_index, edge_type, We, Wih, Whh, bih, bhh)
    emb = _emb_gather_kernel()(tok_pad, emb_table)[:N]
    h = emb
    for _ in range(NSTEPS):
        t4 = _etype_mm(h, wet)
        a = _edge_msg_kernel()(t4.reshape(NC * NETYPES * N, HD), gidx2, dst)
        h = _gru(a, h, wih2, whh, bih2, bhh2)
    sums = _bn_stats(h, emb)
    mean = sums[0] * (1.0 / N)
    var = sums[1] * (1.0 / N) - mean * mean
    mv = jnp.stack([mean, var])
    gb = jnp.stack([gamma, beta])
    return _bn_mean(h, emb, mv, gb)


def kernel(tokens_one, edge_index_one, edge_type_one, tokens_two,
           edge_index_two, edge_type_two, emb_table, We_one, Wih_one, Whh_one,
           bih_one, bhh_one, bn_gamma_one, bn_beta_one, We_two, Wih_two,
           Whh_two, bih_two, bhh_two, bn_gamma_two, bn_beta_two, fW, fb):
    f1 = _ggnn_feats(tokens_one, edge_index_one, edge_type_one, emb_table,
                     We_one, Wih_one, Whh_one, bih_one, bhh_one,
                     bn_gamma_one, bn_beta_one)
    f2 = _ggnn_feats(tokens_two, edge_index_two, edge_type_two, emb_table,
                     We_two, Wih_two, Whh_two, bih_two, bhh_two,
                     bn_gamma_two, bn_beta_two)
    return _head(f1, f2, fW, fb)


# speedup vs baseline: 1.6661x; 1.6661x over previous
import functools

import jax
import jax.numpy as jnp
from jax import lax
from jax.experimental import pallas as pl
from jax.experimental.pallas import tpu as pltpu
from jax.experimental.pallas import tpu_sc as plsc

N = 10000
E = 160000
D = 256
HD = 128
NSTEPS = 3
NETYPES = 3
NC = 2
NS = 16
EPT = E // NS
CHUNK = 200
NCHUNK = EPT // CHUNK
NPAD = 10240
RPT = NPAD // NS
BPW = NPAD // (NC * NS)
NB = 10
BLK = N // NB

@functools.cache
def _mesh():
    return plsc.VectorSubcoreMesh(
        core_axis_name="c", subcore_axis_name="s",
        num_cores=NC, num_subcores=NS)


def _emb_body(tok_hbm, table_hbm, out_hbm, tok_v, rows_v, sem):
    c = lax.axis_index("c")
    s = lax.axis_index("s")
    base = (s * NC + c) * BPW
    pltpu.sync_copy(tok_hbm.at[pl.ds(base, BPW)], tok_v)
    pltpu.async_copy(table_hbm.at[tok_v], rows_v, sem).wait()
    pltpu.sync_copy(rows_v, out_hbm.at[pl.ds(base, BPW)])


@functools.cache
def _emb_gather_kernel():
    return pl.kernel(
        _emb_body,
        out_type=jax.ShapeDtypeStruct((NPAD, D), jnp.float32),
        mesh=_mesh(),
        scratch_types=[
            pltpu.VMEM((BPW,), jnp.int32),
            pltpu.VMEM((BPW, D), jnp.float32),
            pltpu.SemaphoreType.DMA,
        ],
    )


def _edge_body(t_hbm, gidx_hbm, dst_hbm, out_hbm,
               gbuf, dbuf, rows, a_sh, sem):
    c = lax.axis_index("c")
    s = lax.axis_index("s")
    w = c * NS + s

    def zrow(i, carry):
        for j in range(HD // 16):
            rows[i, pl.ds(j * 16, 16)] = jnp.zeros((16,), jnp.float32)
        return carry

    lax.fori_loop(0, 128, zrow, 0)
    for k in range(RPT // 128):
        pltpu.sync_copy(rows.at[pl.ds(0, 128)],
                        a_sh.at[pl.ds(s * RPT + k * 128, 128)])
    plsc.subcore_barrier()

    def chunk(j, carry):
        pltpu.sync_copy(gidx_hbm.at[pl.ds(w * EPT + j * CHUNK, CHUNK)], gbuf)
        pltpu.async_copy(t_hbm.at[gbuf], rows, sem)
        pltpu.sync_copy(dst_hbm.at[pl.ds(s * EPT + j * CHUNK, CHUNK)], dbuf)
        pltpu.make_async_copy(t_hbm.at[gbuf], rows, sem).wait()
        pltpu.sync_copy(rows, a_sh.at[dbuf], add=True)
        return carry

    lax.fori_loop(0, NCHUNK, chunk, 0)

    plsc.subcore_barrier()
    pltpu.sync_copy(a_sh.at[pl.ds(s * RPT, RPT)],
                    out_hbm.at[c, pl.ds(s * RPT, RPT)])


@functools.cache
def _edge_msg_kernel():
    return pl.kernel(
        _edge_body,
        out_type=jax.ShapeDtypeStruct((NC, NPAD, HD), jnp.float32),
        mesh=_mesh(),
        scratch_types=[
            pltpu.VMEM((CHUNK,), jnp.int32),
            pltpu.VMEM((CHUNK,), jnp.int32),
            pltpu.VMEM((CHUNK, HD), jnp.float32),
            pltpu.VMEM_SHARED((NPAD, HD), jnp.float32),
            pltpu.SemaphoreType.DMA,
        ],
    )


def _etype_mm_body(h_ref, wet_ref, out_ref):
    out_ref[0, 0] = jnp.dot(h_ref[...], wet_ref[0],
                            preferred_element_type=jnp.float32)


def _etype_mm(h, wet):
    return pl.pallas_call(
        _etype_mm_body,
        grid=(NB, NC, NETYPES),
        in_specs=[
            pl.BlockSpec((BLK, D), lambda nb, c, e: (nb, 0)),
            pl.BlockSpec((1, D, HD), lambda nb, c, e: (e, 0, c)),
        ],
        out_specs=pl.BlockSpec((1, 1, BLK, HD), lambda nb, c, e: (c, e, nb, 0)),
        out_shape=jax.ShapeDtypeStruct((NC, NETYPES, N, HD), jnp.float32),
    )(h, wet)


def _gru_body(a_ref, h_ref, wih_ref, whh_ref, bih_ref, bhh_ref, out_ref):
    h = h_ref[...]
    gi = (jnp.dot(a_ref[0], wih_ref[0], preferred_element_type=jnp.float32)
          + jnp.dot(a_ref[1], wih_ref[1], preferred_element_type=jnp.float32)
          + bih_ref[0][None, :])
    gh = (jnp.dot(h, whh_ref[...], preferred_element_type=jnp.float32)
          + bhh_ref[0][None, :])
    i_r, i_z, i_n = gi[:, :D], gi[:, D:2 * D], gi[:, 2 * D:]
    h_r, h_z, h_n = gh[:, :D], gh[:, D:2 * D], gh[:, 2 * D:]
    r = 1.0 / (1.0 + jnp.exp(-(i_r + h_r)))
    z = 1.0 / (1.0 + jnp.exp(-(i_z + h_z)))
    n = jnp.tanh(i_n + r * h_n)
    out_ref[...] = (1.0 - z) * n + z * h


def _gru(a, h, wih2, whh, bih, bhh):
    return pl.pallas_call(
        _gru_body,
        grid=(NB,),
        in_specs=[
            pl.BlockSpec((NC, BLK, HD), lambda nb: (0, nb, 0)),
            pl.BlockSpec((BLK, D), lambda nb: (nb, 0)),
            pl.BlockSpec((NC, HD, 3 * D), lambda nb: (0, 0, 0)),
            pl.BlockSpec((D, 3 * D), lambda nb: (0, 0)),
            pl.BlockSpec((1, 3 * D), lambda nb: (0, 0)),
            pl.BlockSpec((1, 3 * D), lambda nb: (0, 0)),
        ],
        out_specs=pl.BlockSpec((BLK, D), lambda nb: (nb, 0)),
        out_shape=jax.ShapeDtypeStruct((N, D), jnp.float32),
    )(a, h, wih2, whh, bih, bhh)


def _bn_stats_body(h_ref, e_ref, out_ref):
    @pl.when(pl.program_id(0) == 0)
    def _init():
        out_ref[...] = jnp.zeros_like(out_ref)

    hb = h_ref[...]
    eb = e_ref[...]
    out_ref[0, :D] += jnp.sum(hb, axis=0)
    out_ref[0, D:] += jnp.sum(eb, axis=0)
    out_ref[1, :D] += jnp.sum(hb * hb, axis=0)
    out_ref[1, D:] += jnp.sum(eb * eb, axis=0)


def _bn_stats(h, emb):
    return pl.pallas_call(
        _bn_stats_body,
        grid=(NB,),
        in_specs=[
            pl.BlockSpec((BLK, D), lambda nb: (nb, 0)),
            pl.BlockSpec((BLK, D), lambda nb: (nb, 0)),
        ],
        out_specs=pl.BlockSpec((2, 2 * D), lambda nb: (0, 0)),
        out_shape=jax.ShapeDtypeStruct((2, 2 * D), jnp.float32),
    )(h, emb)


def _bn_mean_body(h_ref, e_ref, mv_ref, gb_ref, out_ref):
    @pl.when(pl.program_id(0) == 0)
    def _init():
        out_ref[...] = jnp.zeros_like(out_ref)

    inv = lax.rsqrt(mv_ref[1] + 1e-5)
    mean = mv_ref[0]
    xh = (h_ref[...] - mean[:D][None, :]) * inv[:D][None, :]
    xe = (e_ref[...] - mean[D:][None, :]) * inv[D:][None, :]
    out_ref[0, :D] += jnp.sum(xh, axis=0)
    out_ref[0, D:] += jnp.sum(xe, axis=0)

    @pl.when(pl.program_id(0) == NB - 1)
    def _fin():
        out_ref[0, :] = (out_ref[0, :] * gb_ref[0] * (1.0 / N)) + gb_ref[1]


def _bn_mean(h, emb, mv, gb):
    return pl.pallas_call(
        _bn_mean_body,
        grid=(NB,),
        in_specs=[
            pl.BlockSpec((BLK, D), lambda nb: (nb, 0)),
            pl.BlockSpec((BLK, D), lambda nb: (nb, 0)),
            pl.BlockSpec((2, 2 * D), lambda nb: (0, 0)),
            pl.BlockSpec((2, 2 * D), lambda nb: (0, 0)),
        ],
        out_specs=pl.BlockSpec((1, 2 * D), lambda nb: (0, 0)),
        out_shape=jax.ShapeDtypeStruct((1, 2 * D), jnp.float32),
    )(h, emb, mv, gb)


def _head_body(f1_ref, f2_ref, fw_ref, fb_ref, out_ref):
    d = f1_ref[0, :] - f2_ref[0, :]
    euc = d * d
    logits = jnp.sum(euc[None, :] * fw_ref[...], axis=1) + fb_ref[0, :]
    act = jnp.where(logits >= 0, logits, 0.01 * logits)
    m = jnp.max(act)
    e = jnp.exp(act - m)
    out_ref[0, :] = e / jnp.sum(e)


def _head(f1, f2, fW, fb):
    return pl.pallas_call(
        _head_body,
        out_shape=jax.ShapeDtypeStruct((1, fW.shape[0]), jnp.float32),
    )(f1, f2, fW, fb.reshape(1, -1))


def _prep(tokens, edge_index, edge_type, We, Wih, Whh, bih, bhh):
    tok_pad = jnp.concatenate(
        [tokens.astype(jnp.int32), jnp.zeros((NPAD - N,), jnp.int32)])
    src = edge_index[0].astype(jnp.int32)
    dst = edge_index[1].astype(jnp.int32)
    gidx = edge_type.astype(jnp.int32) * N + src
    gidx2 = jnp.concatenate([gidx, gidx + NETYPES * N])
    wet = We.transpose(0, 2, 1)
    wih2 = Wih.T.reshape(NC, HD, 3 * D)
    whh = Whh.T
    return tok_pad, gidx2, dst, wet, wih2, whh, bih.reshape(1, -1), bhh.reshape(1, -1)


def _ggnn_feats(tokens, edge_index, edge_type, emb_table, We, Wih, Whh,
                bih, bhh, gamma, beta):
    tok_pad, gidx2, dst, wet, wih2, whh, bih2, bhh2 = _prep(
        tokens, edge_index, edge_type, We, Wih, Whh, bih, bhh)
    emb = _emb_gather_kernel()(tok_pad, emb_table)[:N]
    h = emb
    for _ in range(NSTEPS):
        t4 = _etype_mm(h, wet)
        a = _edge_msg_kernel()(t4.reshape(NC * NETYPES * N, HD), gidx2, dst)
        h = _gru(a, h, wih2, whh, bih2, bhh2)
    sums = _bn_stats(h, emb)
    mean = sums[0] * (1.0 / N)
    var = sums[1] * (1.0 / N) - mean * mean
    mv = jnp.stack([mean, var])
    gb = jnp.stack([gamma, beta])
    return _bn_mean(h, emb, mv, gb)


def kernel(tokens_one, edge_index_one, edge_type_one, tokens_two,
           edge_index_two, edge_type_two, emb_table, We_one, Wih_one, Whh_one,
           bih_one, bhh_one, bn_gamma_one, bn_beta_one, We_two, Wih_two,
           Whh_two, bih_two, bhh_two, bn_gamma_two, bn_beta_two, fW, fb):
    f1 = _ggnn_feats(tokens_one, edge_index_one, edge_type_one, emb_table,
                     We_one, Wih_one, Whh_one, bih_one, bhh_one,
                     bn_gamma_one, bn_beta_one)
    f2 = _ggnn_feats(tokens_two, edge_index_two, edge_type_two, emb_table,
                     We_two, Wih_two, Whh_two, bih_two, bhh_two,
                     bn_gamma_two, bn_beta_two)
    return _head(f1, f2, fW, fb)


# speedup vs baseline: 1.8720x; 1.1236x over previous
import functools

import jax
import jax.numpy as jnp
from jax import lax
from jax.experimental import pallas as pl
from jax.experimental.pallas import tpu as pltpu
from jax.experimental.pallas import tpu_sc as plsc

N = 10000
E = 160000
D = 256
HD = 128
NSTEPS = 3
NETYPES = 3
NC = 2
NS = 16
EPT = E // NS
CHUNK = 200
NCHUNK = EPT // CHUNK
NPAD = 10240
RPT = NPAD // NS
BPW = NPAD // (NC * NS)
NB = 10
BLK = N // NB

@functools.cache
def _mesh():
    return plsc.VectorSubcoreMesh(
        core_axis_name="c", subcore_axis_name="s",
        num_cores=NC, num_subcores=NS)


def _emb_body(tok_hbm, table_hbm, out_hbm, tok_v, rows_v, sem):
    c = lax.axis_index("c")
    s = lax.axis_index("s")
    base = (s * NC + c) * BPW
    pltpu.sync_copy(tok_hbm.at[pl.ds(base, BPW)], tok_v)
    pltpu.async_copy(table_hbm.at[tok_v], rows_v, sem).wait()
    pltpu.sync_copy(rows_v, out_hbm.at[pl.ds(base, BPW)])


@functools.cache
def _emb_gather_kernel():
    return pl.kernel(
        _emb_body,
        out_type=jax.ShapeDtypeStruct((NPAD, D), jnp.float32),
        mesh=_mesh(),
        scratch_types=[
            pltpu.VMEM((BPW,), jnp.int32),
            pltpu.VMEM((BPW, D), jnp.float32),
            pltpu.SemaphoreType.DMA,
        ],
    )


def _edge_body(t_hbm, gidx_hbm, dst_hbm, out_hbm,
               gbuf, gbuf2, dbuf, rows, a_sh, sem):
    c = lax.axis_index("c")
    s = lax.axis_index("s")
    w = c * NS + s

    def zrow(i, carry):
        for j in range(HD // 16):
            rows[i, pl.ds(j * 16, 16)] = jnp.zeros((16,), jnp.float32)
        return carry

    lax.fori_loop(0, 128, zrow, 0)
    for k in range(RPT // 128):
        pltpu.sync_copy(rows.at[pl.ds(0, 128)],
                        a_sh.at[pl.ds(s * RPT + k * 128, 128)])
    plsc.subcore_barrier()

    def _ld_gidx(j, buf):
        pltpu.sync_copy(gidx_hbm.at[pl.ds(w * EPT + j * CHUNK, CHUNK)], buf)

    def _ld_dst(j):
        pltpu.sync_copy(dst_hbm.at[pl.ds(s * EPT + j * CHUNK, CHUNK)], dbuf)

    _ld_gidx(0, gbuf)

    def pair(jj, carry):
        j0 = 2 * jj
        j1 = 2 * jj + 1
        pltpu.async_copy(t_hbm.at[gbuf], rows, sem)
        _ld_dst(j0)
        _ld_gidx(j1, gbuf2)
        pltpu.make_async_copy(t_hbm.at[gbuf], rows, sem).wait()
        pltpu.sync_copy(rows, a_sh.at[dbuf], add=True)

        pltpu.async_copy(t_hbm.at[gbuf2], rows, sem)
        _ld_dst(j1)

        @pl.when(jj + 1 < NCHUNK // 2)
        def _():
            _ld_gidx(j0 + 2, gbuf)

        pltpu.make_async_copy(t_hbm.at[gbuf2], rows, sem).wait()
        pltpu.sync_copy(rows, a_sh.at[dbuf], add=True)
        return carry

    lax.fori_loop(0, NCHUNK // 2, pair, 0)

    plsc.subcore_barrier()
    pltpu.sync_copy(a_sh.at[pl.ds(s * RPT, RPT)],
                    out_hbm.at[c, pl.ds(s * RPT, RPT)])


@functools.cache
def _edge_msg_kernel():
    return pl.kernel(
        _edge_body,
        out_type=jax.ShapeDtypeStruct((NC, NPAD, HD), jnp.float32),
        mesh=_mesh(),
        scratch_types=[
            pltpu.VMEM((CHUNK,), jnp.int32),
            pltpu.VMEM((CHUNK,), jnp.int32),
            pltpu.VMEM((CHUNK,), jnp.int32),
            pltpu.VMEM((CHUNK, HD), jnp.float32),
            pltpu.VMEM_SHARED((NPAD, HD), jnp.float32),
            pltpu.SemaphoreType.DMA,
        ],
    )


def _etype_mm_body(h_ref, wet_ref, out_ref):
    out_ref[0, 0] = jnp.dot(h_ref[...], wet_ref[0],
                            preferred_element_type=jnp.float32)


def _etype_mm(h, wet):
    return pl.pallas_call(
        _etype_mm_body,
        grid=(NB, NC, NETYPES),
        in_specs=[
            pl.BlockSpec((BLK, D), lambda nb, c, e: (nb, 0)),
            pl.BlockSpec((1, D, HD), lambda nb, c, e: (e, 0, c)),
        ],
        out_specs=pl.BlockSpec((1, 1, BLK, HD), lambda nb, c, e: (c, e, nb, 0)),
        out_shape=jax.ShapeDtypeStruct((NC, NETYPES, N, HD), jnp.float32),
    )(h, wet)


def _gru_body(a_ref, h_ref, wih_ref, whh_ref, bih_ref, bhh_ref, out_ref):
    h = h_ref[...]
    gi = (jnp.dot(a_ref[0], wih_ref[0], preferred_element_type=jnp.float32)
          + jnp.dot(a_ref[1], wih_ref[1], preferred_element_type=jnp.float32)
          + bih_ref[0][None, :])
    gh = (jnp.dot(h, whh_ref[...], preferred_element_type=jnp.float32)
          + bhh_ref[0][None, :])
    i_r, i_z, i_n = gi[:, :D], gi[:, D:2 * D], gi[:, 2 * D:]
    h_r, h_z, h_n = gh[:, :D], gh[:, D:2 * D], gh[:, 2 * D:]
    r = 1.0 / (1.0 + jnp.exp(-(i_r + h_r)))
    z = 1.0 / (1.0 + jnp.exp(-(i_z + h_z)))
    n = jnp.tanh(i_n + r * h_n)
    out_ref[...] = (1.0 - z) * n + z * h


def _gru(a, h, wih2, whh, bih, bhh):
    return pl.pallas_call(
        _gru_body,
        grid=(NB,),
        in_specs=[
            pl.BlockSpec((NC, BLK, HD), lambda nb: (0, nb, 0)),
            pl.BlockSpec((BLK, D), lambda nb: (nb, 0)),
            pl.BlockSpec((NC, HD, 3 * D), lambda nb: (0, 0, 0)),
            pl.BlockSpec((D, 3 * D), lambda nb: (0, 0)),
            pl.BlockSpec((1, 3 * D), lambda nb: (0, 0)),
            pl.BlockSpec((1, 3 * D), lambda nb: (0, 0)),
        ],
        out_specs=pl.BlockSpec((BLK, D), lambda nb: (nb, 0)),
        out_shape=jax.ShapeDtypeStruct((N, D), jnp.float32),
    )(a, h, wih2, whh, bih, bhh)


def _bn_stats_body(h_ref, e_ref, out_ref):
    @pl.when(pl.program_id(0) == 0)
    def _init():
        out_ref[...] = jnp.zeros_like(out_ref)

    hb = h_ref[...]
    eb = e_ref[...]
    out_ref[0, :D] += jnp.sum(hb, axis=0)
    out_ref[0, D:] += jnp.sum(eb, axis=0)
    out_ref[1, :D] += jnp.sum(hb * hb, axis=0)
    out_ref[1, D:] += jnp.sum(eb * eb, axis=0)


def _bn_stats(h, emb):
    return pl.pallas_call(
        _bn_stats_body,
        grid=(NB,),
        in_specs=[
            pl.BlockSpec((BLK, D), lambda nb: (nb, 0)),
            pl.BlockSpec((BLK, D), lambda nb: (nb, 0)),
        ],
        out_specs=pl.BlockSpec((2, 2 * D), lambda nb: (0, 0)),
        out_shape=jax.ShapeDtypeStruct((2, 2 * D), jnp.float32),
    )(h, emb)


def _bn_mean_body(h_ref, e_ref, mv_ref, gb_ref, out_ref):
    @pl.when(pl.program_id(0) == 0)
    def _init():
        out_ref[...] = jnp.zeros_like(out_ref)

    inv = lax.rsqrt(mv_ref[1] + 1e-5)
    mean = mv_ref[0]
    xh = (h_ref[...] - mean[:D][None, :]) * inv[:D][None, :]
    xe = (e_ref[...] - mean[D:][None, :]) * inv[D:][None, :]
    out_ref[0, :D] += jnp.sum(xh, axis=0)
    out_ref[0, D:] += jnp.sum(xe, axis=0)

    @pl.when(pl.program_id(0) == NB - 1)
    def _fin():
        out_ref[0, :] = (out_ref[0, :] * gb_ref[0] * (1.0 / N)) + gb_ref[1]


def _bn_mean(h, emb, mv, gb):
    return pl.pallas_call(
        _bn_mean_body,
        grid=(NB,),
        in_specs=[
            pl.BlockSpec((BLK, D), lambda nb: (nb, 0)),
            pl.BlockSpec((BLK, D), lambda nb: (nb, 0)),
            pl.BlockSpec((2, 2 * D), lambda nb: (0, 0)),
            pl.BlockSpec((2, 2 * D), lambda nb: (0, 0)),
        ],
        out_specs=pl.BlockSpec((1, 2 * D), lambda nb: (0, 0)),
        out_shape=jax.ShapeDtypeStruct((1, 2 * D), jnp.float32),
    )(h, emb, mv, gb)


def _head_body(f1_ref, f2_ref, fw_ref, fb_ref, out_ref):
    d = f1_ref[0, :] - f2_ref[0, :]
    euc = d * d
    logits = jnp.sum(euc[None, :] * fw_ref[...], axis=1) + fb_ref[0, :]
    act = jnp.where(logits >= 0, logits, 0.01 * logits)
    m = jnp.max(act)
    e = jnp.exp(act - m)
    out_ref[0, :] = e / jnp.sum(e)


def _head(f1, f2, fW, fb):
    return pl.pallas_call(
        _head_body,
        out_shape=jax.ShapeDtypeStruct((1, fW.shape[0]), jnp.float32),
    )(f1, f2, fW, fb.reshape(1, -1))


def _prep(tokens, edge_index, edge_type, We, Wih, Whh, bih, bhh):
    tok_pad = jnp.concatenate(
        [tokens.astype(jnp.int32), jnp.zeros((NPAD - N,), jnp.int32)])
    src = edge_index[0].astype(jnp.int32)
    dst = edge_index[1].astype(jnp.int32)
    gidx = edge_type.astype(jnp.int32) * N + src
    gidx2 = jnp.concatenate([gidx, gidx + NETYPES * N])
    wet = We.transpose(0, 2, 1)
    wih2 = Wih.T.reshape(NC, HD, 3 * D)
    whh = Whh.T
    return tok_pad, gidx2, dst, wet, wih2, whh, bih.reshape(1, -1), bhh.reshape(1, -1)


def _ggnn_feats(tokens, edge_index, edge_type, emb_table, We, Wih, Whh,
                bih, bhh, gamma, beta):
    tok_pad, gidx2, dst, wet, wih2, whh, bih2, bhh2 = _prep(
        tokens, edge_index, edge_type, We, Wih, Whh, bih, bhh)
    emb = _emb_gather_kernel()(tok_pad, emb_table)[:N]
    h = emb
    for _ in range(NSTEPS):
        t4 = _etype_mm(h, wet)
        a = _edge_msg_kernel()(t4.reshape(NC * NETYPES * N, HD), gidx2, dst)
        h = _gru(a, h, wih2, whh, bih2, bhh2)
    sums = _bn_stats(h, emb)
    mean = sums[0] * (1.0 / N)
    var = sums[1] * (1.0 / N) - mean * mean
    mv = jnp.stack([mean, var])
    gb = jnp.stack([gamma, beta])
    return _bn_mean(h, emb, mv, gb)


def kernel(tokens_one, edge_index_one, edge_type_one, tokens_two,
           edge_index_two, edge_type_two, emb_table, We_one, Wih_one, Whh_one,
           bih_one, bhh_one, bn_gamma_one, bn_beta_one, We_two, Wih_two,
           Whh_two, bih_two, bhh_two, bn_gamma_two, bn_beta_two, fW, fb):
    f1 = _ggnn_feats(tokens_one, edge_index_one, edge_type_one, emb_table,
                     We_one, Wih_one, Whh_one, bih_one, bhh_one,
                     bn_gamma_one, bn_beta_one)
    f2 = _ggnn_feats(tokens_two, edge_index_two, edge_type_two, emb_table,
                     We_two, Wih_two, Whh_two, bih_two, bhh_two,
                     bn_gamma_two, bn_beta_two)
    return _head(f1, f2, fW, fb)
